# baseline (speedup 1.0000x reference)
import jax
import jax.numpy as jnp
from jax import lax
from jax.experimental import pallas as pl
from jax.experimental.pallas import tpu as pltpu
from jax.experimental.pallas import tpu_sc as plsc

VSTART = 0
VEND = 250000
D = 64
NC = 2
NS = 16
NW = NC * NS
NB = 16
L = 50
C = NB * L
SUB = 128
NSUB = (C + SUB - 1) // SUB

_DNUMS = lax.GatherDimensionNumbers(
    offset_dims=(), collapsed_slice_dims=(0,), start_index_map=(0,)
)


def _splat0(vec):
    lane0 = jnp.reshape(lax.iota(jnp.int32, 16) * 0, (16, 1))
    return lax.gather(
        vec, lane0, _DNUMS, (1,), mode=lax.GatherScatterMode.PROMISE_IN_BOUNDS
    )


def _body(idx_hbm, w_hbm, out_hbm, idxv, cidxv, clocv, cgat, stag,
          semg, semo):
    wid = lax.axis_index("s") * NC + lax.axis_index("c")
    n_b = out_hbm.shape[0]
    b_per_w = n_b // NW
    n_chunks = b_per_w // NB
    base_b = wid * b_per_w

    def chunk(k, carry):
        b0 = base_b + k * NB
        row0 = b0 * L
        pltpu.sync_copy(idx_hbm.at[pl.ds(row0, C)], idxv)

        def cpass(i, c):
            v = idxv[pl.ds(i * 16, 16)]
            loc = lax.iota(jnp.int32, 16) + i * 16
            m = (v >= VSTART) & (v < VEND)
            plsc.store_compressed(cidxv.at[pl.ds(c, 16)], v - VSTART, mask=m)
            plsc.store_compressed(clocv.at[pl.ds(c, 16)], loc, mask=m)
            return c + jnp.max(plsc.all_reduce_population_count(m))

        nc = lax.fori_loop(0, C // 16, cpass, 0)
        nc_pad = ((nc + SUB - 1) // SUB) * SUB

        spl_i = _splat0(cidxv[pl.ds(0, 16)])
        spl_p = _splat0(clocv[pl.ds(0, 16)])

        def fill(i, carry2):
            @pl.when(nc + i * 16 < nc_pad)
            def _():
                cidxv[pl.ds(nc + i * 16, 16)] = spl_i
                clocv[pl.ds(nc + i * 16, 16)] = spl_p

            return carry2

        lax.fori_loop(0, SUB // 16, fill, 0)

        for j in range(NSUB):
            @pl.when(nc > j * SUB)
            def _(j=j):
                pltpu.async_copy(
                    w_hbm.at[cidxv.at[pl.ds(j * SUB, SUB)]],
                    cgat.at[pl.ds(j * SUB, SUB)],
                    semg,
                )
        def zs(g, carry2):
            for q in range(4):
                r = g * 4 + q
                for c4 in range(4):
                    stag[r, pl.ds(c4 * 16, 16)] = jnp.zeros((16,), jnp.float32)
            return carry2

        lax.fori_loop(0, C // 4, zs, 0)

        for j in range(NSUB):
            @pl.when(nc > j * SUB)
            def _(j=j):
                pltpu.make_async_copy(
                    w_hbm.at[cidxv.at[pl.ds(j * SUB, SUB)]],
                    cgat.at[pl.ds(j * SUB, SUB)],
                    semg,
                ).wait()
        def expand(g, carry2):
            lvec = clocv[pl.ds(g * 16, 16)]
            for rr in range(16):
                loc = lvec[rr]
                for c4 in range(4):
                    sl = pl.ds(c4 * 16, 16)
                    stag[loc, sl] = cgat[g * 16 + rr, sl]
            return carry2

        lax.fori_loop(0, nc_pad // 16, expand, 0)

        for s in range(NB):
            pltpu.async_copy(
                stag.at[pl.ds(s * L, L)], out_hbm.at[b0 + s], semo
            )
        for s in range(NB):
            pltpu.make_async_copy(
                stag.at[pl.ds(s * L, L)], out_hbm.at[b0 + s], semo
            ).wait()

        return carry

    lax.fori_loop(0, n_chunks, chunk, 0)


def kernel(input, weight):
    b, l = input.shape
    nflat = b * l
    idx = input.reshape(nflat).astype(jnp.int32)
    mesh = plsc.VectorSubcoreMesh(core_axis_name="c", subcore_axis_name="s")
    out = pl.kernel(
        _body,
        out_type=jax.ShapeDtypeStruct((b, l, D), jnp.float32),
        mesh=mesh,
        compiler_params=pltpu.CompilerParams(
            use_tc_tiling_on_sc=False, needs_layout_passes=False
        ),
        scratch_types=[
            pltpu.VMEM((C,), jnp.int32),
            pltpu.VMEM((NSUB * SUB + 16,), jnp.int32),
            pltpu.VMEM((NSUB * SUB + 16,), jnp.int32),
            pltpu.VMEM((NSUB * SUB, D), jnp.float32),
            pltpu.VMEM((C, D), jnp.float32),
            pltpu.SemaphoreType.DMA,
            pltpu.SemaphoreType.DMA,
        ],
    )(idx, weight)
    return out


# speedup vs baseline: 1.0544x; 1.0544x over previous
import jax
import jax.numpy as jnp
from jax import lax
from jax.experimental import pallas as pl
from jax.experimental.pallas import tpu as pltpu
from jax.experimental.pallas import tpu_sc as plsc

VSTART = 0
VEND = 250000
D = 64
NC = 2
NS = 16
NW = NC * NS
NB = 16
L = 50
C = NB * L
SUB = 128
NSUB = (C + SUB - 1) // SUB

_DNUMS = lax.GatherDimensionNumbers(
    offset_dims=(), collapsed_slice_dims=(0,), start_index_map=(0,)
)


def _splat0(vec):
    lane0 = jnp.reshape(lax.iota(jnp.int32, 16) * 0, (16, 1))
    return lax.gather(
        vec, lane0, _DNUMS, (1,), mode=lax.GatherScatterMode.PROMISE_IN_BOUNDS
    )


def _body(idx_hbm, w_hbm, out_hbm, idxv, cidxv, clocv, cgat, stag,
          semg, semo):
    wid = lax.axis_index("s") * NC + lax.axis_index("c")
    n_b = out_hbm.shape[1]
    b_per_w = n_b // NW
    n_chunks = b_per_w // NB
    base_b = wid * b_per_w

    def chunk(k, carry):
        b0 = base_b + k * NB
        row0 = b0 * L
        pltpu.sync_copy(idx_hbm.at[pl.ds(row0, C)], idxv)

        def cpass(i, c):
            v = idxv[pl.ds(i * 16, 16)]
            f = lax.iota(jnp.int32, 16) + i * 16
            db = f // L
            loc = (f - db * L) * NB + db
            m = (v >= VSTART) & (v < VEND)
            plsc.store_compressed(cidxv.at[pl.ds(c, 16)], v - VSTART, mask=m)
            plsc.store_compressed(clocv.at[pl.ds(c, 16)], loc, mask=m)
            return c + jnp.max(plsc.all_reduce_population_count(m))

        nc = lax.fori_loop(0, C // 16, cpass, 0)
        nc_pad = ((nc + SUB - 1) // SUB) * SUB

        spl_i = _splat0(cidxv[pl.ds(0, 16)])
        spl_p = _splat0(clocv[pl.ds(0, 16)])

        def fill(i, carry2):
            @pl.when(nc + i * 16 < nc_pad)
            def _():
                cidxv[pl.ds(nc + i * 16, 16)] = spl_i
                clocv[pl.ds(nc + i * 16, 16)] = spl_p

            return carry2

        lax.fori_loop(0, SUB // 16, fill, 0)

        for j in range(NSUB):
            @pl.when(nc > j * SUB)
            def _(j=j):
                pltpu.async_copy(
                    w_hbm.at[cidxv.at[pl.ds(j * SUB, SUB)]],
                    cgat.at[pl.ds(j * SUB, SUB)],
                    semg,
                )
        def zs(g, carry2):
            for q in range(4):
                r = g * 4 + q
                for c4 in range(4):
                    stag[r, pl.ds(c4 * 16, 16)] = jnp.zeros((16,), jnp.float32)
            return carry2

        lax.fori_loop(0, C // 4, zs, 0)

        for j in range(NSUB):
            @pl.when(nc > j * SUB)
            def _(j=j):
                pltpu.make_async_copy(
                    w_hbm.at[cidxv.at[pl.ds(j * SUB, SUB)]],
                    cgat.at[pl.ds(j * SUB, SUB)],
                    semg,
                ).wait()
        def expand(g, carry2):
            lvec = clocv[pl.ds(g * 16, 16)]
            for rr in range(16):
                loc = lvec[rr]
                for c4 in range(4):
                    sl = pl.ds(c4 * 16, 16)
                    stag[loc, sl] = cgat[g * 16 + rr, sl]
            return carry2

        lax.fori_loop(0, nc_pad // 16, expand, 0)

        for s in range(L):
            pltpu.async_copy(
                stag.at[pl.ds(s * NB, NB)], out_hbm.at[s, pl.ds(b0, NB)], semo
            )
        for s in range(L):
            pltpu.make_async_copy(
                stag.at[pl.ds(s * NB, NB)], out_hbm.at[s, pl.ds(b0, NB)], semo
            ).wait()

        return carry

    lax.fori_loop(0, n_chunks, chunk, 0)


def kernel(input, weight):
    b, l = input.shape
    nflat = b * l
    idx = input.reshape(nflat).astype(jnp.int32)
    mesh = plsc.VectorSubcoreMesh(core_axis_name="c", subcore_axis_name="s")
    out = pl.kernel(
        _body,
        out_type=jax.ShapeDtypeStruct((l, b, D), jnp.float32),
        mesh=mesh,
        compiler_params=pltpu.CompilerParams(
            use_tc_tiling_on_sc=False, needs_layout_passes=False
        ),
        scratch_types=[
            pltpu.VMEM((C,), jnp.int32),
            pltpu.VMEM((NSUB * SUB + 16,), jnp.int32),
            pltpu.VMEM((NSUB * SUB + 16,), jnp.int32),
            pltpu.VMEM((NSUB * SUB, D), jnp.float32),
            pltpu.VMEM((C, D), jnp.float32),
            pltpu.SemaphoreType.DMA,
            pltpu.SemaphoreType.DMA,
        ],
    )(idx, weight)
    return jnp.transpose(out, (1, 0, 2))
